# baseline (speedup 1.0000x reference)
import jax
import jax.numpy as jnp
from jax.experimental import pallas as pl
from jax.experimental.pallas import tpu as pltpu
from jax.experimental.pallas import tpu_sc as plsc

LATENT = 256
HID = 512
NL = 8
NPD = 2
NEXP = NPD ** 3
IN_DIM = 3
OUT_DIM = 1

SEG = 128
TM = 512
TP = 16384


def _vector_mesh():
    return plsc.VectorSubcoreMesh(core_axis_name="c", subcore_axis_name="s")


def _sc_reorder(data, pos_cur, out_rows, pos_prev=None):
    T = pos_cur.shape[1]
    D = data.shape[1]
    WIN = 64
    NSUB = 32
    PER = T // NSUB
    NW = PER // WIN
    indexed = pos_prev is not None

    def body(*args):
        if indexed:
            data_hbm, pc_hbm, pp_hbm, o_hbm, buf, pidx, gsem, ssem = args
        else:
            data_hbm, pc_hbm, o_hbm, buf, pidx, gsem, ssem = args
        c = jax.lax.axis_index("c")
        s = jax.lax.axis_index("s")
        base = (c * 16 + s) * PER
        pltpu.sync_copy(pc_hbm.at[0, pl.ds(base, PER)], pidx.at[0])
        if indexed:
            pltpu.sync_copy(pp_hbm.at[0, pl.ds(base, PER)], pidx.at[1])

        def gather_copy(w):
            if indexed:
                src = data_hbm.at[pidx.at[1, pl.ds(w * WIN, WIN)]]
            else:
                src = data_hbm.at[pl.ds(base + w * WIN, WIN), :]
            return pltpu.make_async_copy(src, buf.at[w % 2], gsem.at[w % 2])

        def scatter_copy(w):
            dst = o_hbm.at[pidx.at[0, pl.ds(w * WIN, WIN)]]
            return pltpu.make_async_copy(buf.at[w % 2], dst, ssem.at[w % 2])

        g = [gather_copy(w) for w in range(NW)]
        sc = [scatter_copy(w) for w in range(NW)]
        g[0].start()
        if NW > 1:
            g[1].start()
        for w in range(NW):
            g[w].wait()
            sc[w].start()
            if w + 2 < NW:
                sc[w].wait()
                g[w + 2].start()
        for w in range(max(0, NW - 2), NW):
            sc[w].wait()

    scratch = [pltpu.VMEM((2, WIN, D), data.dtype),
               pltpu.VMEM((2 if indexed else 1, PER), jnp.int32),
               pltpu.SemaphoreType.DMA((2,)),
               pltpu.SemaphoreType.DMA((2,))]
    k = pl.kernel(body,
                  out_type=jax.ShapeDtypeStruct((out_rows, D), data.dtype),
                  mesh=_vector_mesh(),
                  scratch_types=scratch)
    if indexed:
        return k(data, pos_cur, pos_prev)
    return k(data, pos_cur)


def _sc_permute(data, pos_prev, pos_cur, out_rows):
    return _sc_reorder(data, pos_cur, out_rows, pos_prev=pos_prev)


def _sc_scatter(data, pos_cur, out_rows):
    return _sc_reorder(data, pos_cur, out_rows)


def _sc_scatter2(data, pos_a, pos_b, out_rows):
    T = pos_a.shape[1]
    D = data.shape[1]
    WIN = 64
    NSUB = 32
    PER = T // NSUB
    NW = PER // WIN

    def body(data_hbm, pa_hbm, pb_hbm, oa_hbm, ob_hbm,
             buf, pidx, gsem, asem, bsem):
        c = jax.lax.axis_index("c")
        s = jax.lax.axis_index("s")
        base = (c * 16 + s) * PER
        pltpu.sync_copy(pa_hbm.at[0, pl.ds(base, PER)], pidx.at[0])
        pltpu.sync_copy(pb_hbm.at[0, pl.ds(base, PER)], pidx.at[1])

        def gather_copy(w):
            src = data_hbm.at[pl.ds(base + w * WIN, WIN), :]
            return pltpu.make_async_copy(src, buf.at[w % 2], gsem.at[w % 2])

        def scatter_copy(w, o_hbm, row, sem):
            dst = o_hbm.at[pidx.at[row, pl.ds(w * WIN, WIN)]]
            return pltpu.make_async_copy(buf.at[w % 2], dst, sem.at[w % 2])

        g = [gather_copy(w) for w in range(NW)]
        sa = [scatter_copy(w, oa_hbm, 0, asem) for w in range(NW)]
        sb = [scatter_copy(w, ob_hbm, 1, bsem) for w in range(NW)]
        g[0].start()
        if NW > 1:
            g[1].start()
        for w in range(NW):
            g[w].wait()
            sa[w].start()
            sb[w].start()
            if w + 2 < NW:
                sa[w].wait()
                sb[w].wait()
                g[w + 2].start()
        for w in range(max(0, NW - 2), NW):
            sa[w].wait()
            sb[w].wait()

    out = jax.ShapeDtypeStruct((out_rows, D), data.dtype)
    scratch = [pltpu.VMEM((2, WIN, D), data.dtype),
               pltpu.VMEM((2, PER), jnp.int32),
               pltpu.SemaphoreType.DMA((2,)),
               pltpu.SemaphoreType.DMA((2,)),
               pltpu.SemaphoreType.DMA((2,))]
    k = pl.kernel(body, out_type=[out, out],
                  mesh=_vector_mesh(), scratch_types=scratch)
    return k(data, pos_a, pos_b)


def _sc_gather(data, pos):
    T = pos.shape[1]
    D = data.shape[1]
    WIN = 128

    @pl.kernel(out_type=jax.ShapeDtypeStruct((T, D), data.dtype),
               mesh=_vector_mesh())
    def k(data_hbm, p_hbm, o_hbm):
        def body(p_vmem, o_vmem):
            pltpu.sync_copy(data_hbm.at[p_vmem.at[0]], o_vmem)

        pltpu.emit_pipeline(
            body,
            grid=(T // WIN,),
            in_specs=[pl.BlockSpec((1, WIN), lambda i: (0, i))],
            out_specs=[pl.BlockSpec((WIN, D), lambda i: (i, 0))],
            core_axis_name=("c", "s"),
            dimension_semantics=(pltpu.PARALLEL,),
        )(p_hbm, o_hbm)

    return k(data, pos)


def _tc_pair(A_list, WA_list, bA, WB, bB, meta, reluB):
    NA = WA_list[0].shape[2]
    NB = WB.shape[2]
    nA = len(A_list)
    nb = TP // TM
    nsub = TM // SEG

    def body(*refs):
        eA_r, eB_r, act_r = refs[:3]
        a_refs = refs[3:3 + nA]
        wa_refs = refs[3 + nA:3 + 2 * nA]
        bA_r, wB_r, bB_r, o_ref = refs[3 + 2 * nA:]
        m = pl.program_id(0)

        for sub in range(nsub):
            s = m * nsub + sub
            rows = pl.ds(sub * SEG, SEG)

            @pl.when(act_r[s] == 1)
            def _():
                eA = eA_r[s]
                eB = eB_r[s]
                a0 = a_refs[0][rows, :].astype(jnp.bfloat16)
                acc = jnp.dot(a0, wa_refs[0][eA],
                              preferred_element_type=jnp.float32)
                for a_r, w_r in zip(a_refs[1:], wa_refs[1:]):
                    aj = a_r[rows, :].astype(jnp.bfloat16)
                    acc = acc + jnp.dot(aj, w_r[eA],
                                        preferred_element_type=jnp.float32)
                hA = jnp.maximum(acc + bA_r[...], 0.0).astype(jnp.bfloat16)
                out = jnp.dot(hA, wB_r[eB],
                              preferred_element_type=jnp.float32)
                out = out + bB_r[...]
                if reluB:
                    out = jnp.maximum(out, 0.0)
                o_ref[rows, :] = out.astype(o_ref.dtype)

    in_specs = []
    for A in A_list:
        K = A.shape[1]
        in_specs.append(pl.BlockSpec((TM, K), lambda m, *s: (m, 0)))
    for W in WA_list:
        in_specs.append(pl.BlockSpec(W.shape, lambda m, *s: (0, 0, 0)))
    in_specs.append(pl.BlockSpec((1, NA), lambda m, *s: (0, 0)))
    in_specs.append(pl.BlockSpec(WB.shape, lambda m, *s: (0, 0, 0)))
    in_specs.append(pl.BlockSpec((1, NB), lambda m, *s: (0, 0)))

    grid_spec = pltpu.PrefetchScalarGridSpec(
        num_scalar_prefetch=3,
        grid=(nb,),
        in_specs=in_specs,
        out_specs=pl.BlockSpec((TM, NB), lambda m, *s: (m, 0)),
    )
    return pl.pallas_call(
        body,
        grid_spec=grid_spec,
        out_shape=jax.ShapeDtypeStruct((TP, NB), jnp.float32),
    )(meta["eidA"], meta["eidB"], meta["act"],
      *A_list, *WA_list, bA, WB, bB)


def _routing_pairs(xyz_f):
    T = xyz_f.shape[0]
    NBK = NEXP * NEXP
    BLK = 128
    NBLK = T // BLK
    nsub = TP // SEG
    tril = jnp.tril(jnp.ones((BLK, BLK), jnp.float32), k=-1)
    bids = jnp.arange(NBK, dtype=jnp.int32)
    s_starts = jnp.arange(nsub, dtype=jnp.int32) * SEG
    metas = []
    for p in range(NL // 2):
        tids = []
        for i in (2 * p, 2 * p + 1):
            alpha = 2.0 ** (i + 1)
            t3 = jnp.floor(alpha * xyz_f).astype(jnp.int32) % NPD
            tids.append(t3[:, 0] + NPD * t3[:, 1] + NPD ** 2 * t3[:, 2])
        key = tids[0] * NEXP + tids[1]
        oh = (key[:, None] == bids[None, :]).astype(jnp.float32)
        oh3 = oh.reshape(NBLK, BLK, NBK)
        intra = jnp.einsum("lk,bkc->blc", tril, oh3)
        bs = jnp.sum(oh3, axis=1)
        blockoff = jnp.cumsum(bs, axis=0) - bs
        counts = jnp.sum(bs, axis=0).astype(jnp.int32)
        pc = (counts + SEG - 1) // SEG * SEG
        pcum = jnp.cumsum(pc)
        poffs = (pcum - pc).astype(jnp.float32)
        pos3 = intra + blockoff[:, None, :] + poffs[None, None, :]
        pos = jnp.sum(pos3 * oh3, axis=2).reshape(T).astype(jnp.int32)
        bk = jnp.sum(
            (s_starts[:, None] >= pcum[None, :]).astype(jnp.int32), axis=1)
        act = (s_starts < pcum[-1]).astype(jnp.int32)
        bk = jnp.minimum(bk, NBK - 1)
        metas.append(dict(
            pos=pos.reshape(1, T),
            eidA=bk // NEXP,
            eidB=bk % NEXP,
            act=act))
    return metas


def _to32(a):
    T, D = a.shape
    return jax.lax.bitcast_convert_type(
        a.reshape(T, D // 2, 2), jnp.int32)


def _from32(a, D):
    return jax.lax.bitcast_convert_type(
        a, jnp.bfloat16).reshape(a.shape[0], D)


def _pad_cols(a, to):
    pad = to - a.shape[-1]
    if pad == 0:
        return a
    cfg = [(0, 0)] * (a.ndim - 1) + [(0, pad)]
    return jnp.pad(a, cfg)


def _pad_rows(w, to):
    pad = to - w.shape[1]
    if pad == 0:
        return w
    return jnp.pad(w, [(0, 0), (0, pad), (0, 0)])


def kernel(lat, xyz, W0, W1, W2, W3, W4, W5, W6, W7,
           b0, b1, b2, b3, b4, b5, b6, b7):
    B, N, _ = xyz.shape
    T = B * N
    batch_shape = xyz.shape[:-1]
    XF = LATENT + IN_DIM
    XP = 384
    SKIP = HID - XF
    SKIPP = 256

    xyz_f = xyz.reshape(T, IN_DIM)
    x = jnp.concatenate(
        [jnp.broadcast_to(lat, batch_shape + (LATENT,)), xyz],
        axis=-1).reshape(T, XF)
    x = _pad_cols(x, XP)

    m0, m1, m2, m3 = _routing_pairs(xyz_f)

    bf = jnp.bfloat16
    W0p = _pad_rows(W0, XP).astype(bf)
    W1b = W1.astype(bf)
    W2b = W2.astype(bf)
    W3p = _pad_cols(W3, SKIPP).astype(bf)
    b3p = _pad_cols(b3, SKIPP)
    W4a = _pad_rows(W4[:, :SKIP, :], SKIPP).astype(bf)
    W4b = _pad_rows(W4[:, SKIP:, :], XP).astype(bf)
    W5b = W5.astype(bf)
    W6b = W6.astype(bf)
    W7p = _pad_cols(W7, 128).astype(bf)
    b7p = _pad_cols(b7, 128)

    x_s0, x_s4 = _sc_scatter2(x, m0["pos"], m2["pos"], TP)
    h1 = _tc_pair([x_s0], [W0p], b0, W1b, b1, m0, reluB=True)
    h1p = _sc_permute(h1, m0["pos"], m1["pos"], TP)
    h3 = _tc_pair([h1p], [W2b], b2, W3p, b3p, m1, reluB=True)
    h3p = _sc_permute(h3, m1["pos"], m2["pos"], TP)
    h5 = _tc_pair([h3p, x_s4], [W4a, W4b], b4, W5b, b5, m2, reluB=True)
    h5p = _sc_permute(h5, m2["pos"], m3["pos"], TP)
    out7 = _tc_pair([h5p], [W6b], b6, W7p, b7p, m3, reluB=False)
    y = _sc_gather(out7, m3["pos"])
    return y[:, :OUT_DIM].astype(jnp.float32).reshape(
        batch_shape + (OUT_DIM,))


# speedup vs baseline: 1.0324x; 1.0324x over previous
import jax
import jax.numpy as jnp
from jax.experimental import pallas as pl
from jax.experimental.pallas import tpu as pltpu
from jax.experimental.pallas import tpu_sc as plsc

LATENT = 256
HID = 512
NL = 8
NPD = 2
NEXP = NPD ** 3
IN_DIM = 3
OUT_DIM = 1

SEG = 128
TM = 1024
TP = 16384


def _vector_mesh():
    return plsc.VectorSubcoreMesh(core_axis_name="c", subcore_axis_name="s")


def _sc_reorder(data, pos_cur, out_rows, pos_prev=None):
    T = pos_cur.shape[1]
    D = data.shape[1]
    WIN = 64
    NSUB = 32
    PER = T // NSUB
    NW = PER // WIN
    indexed = pos_prev is not None

    def body(*args):
        if indexed:
            data_hbm, pc_hbm, pp_hbm, o_hbm, buf, pidx, gsem, ssem = args
        else:
            data_hbm, pc_hbm, o_hbm, buf, pidx, gsem, ssem = args
        c = jax.lax.axis_index("c")
        s = jax.lax.axis_index("s")
        base = (c * 16 + s) * PER
        pltpu.sync_copy(pc_hbm.at[0, pl.ds(base, PER)], pidx.at[0])
        if indexed:
            pltpu.sync_copy(pp_hbm.at[0, pl.ds(base, PER)], pidx.at[1])

        def gather_copy(w):
            if indexed:
                src = data_hbm.at[pidx.at[1, pl.ds(w * WIN, WIN)]]
            else:
                src = data_hbm.at[pl.ds(base + w * WIN, WIN), :]
            return pltpu.make_async_copy(src, buf.at[w % 2], gsem.at[w % 2])

        def scatter_copy(w):
            dst = o_hbm.at[pidx.at[0, pl.ds(w * WIN, WIN)]]
            return pltpu.make_async_copy(buf.at[w % 2], dst, ssem.at[w % 2])

        g = [gather_copy(w) for w in range(NW)]
        sc = [scatter_copy(w) for w in range(NW)]
        g[0].start()
        if NW > 1:
            g[1].start()
        for w in range(NW):
            g[w].wait()
            sc[w].start()
            if w + 2 < NW:
                sc[w].wait()
                g[w + 2].start()
        for w in range(max(0, NW - 2), NW):
            sc[w].wait()

    scratch = [pltpu.VMEM((2, WIN, D), data.dtype),
               pltpu.VMEM((2 if indexed else 1, PER), jnp.int32),
               pltpu.SemaphoreType.DMA((2,)),
               pltpu.SemaphoreType.DMA((2,))]
    k = pl.kernel(body,
                  out_type=jax.ShapeDtypeStruct((out_rows, D), data.dtype),
                  mesh=_vector_mesh(),
                  scratch_types=scratch)
    if indexed:
        return k(data, pos_cur, pos_prev)
    return k(data, pos_cur)


def _sc_permute(data, pos_prev, pos_cur, out_rows):
    return _sc_reorder(data, pos_cur, out_rows, pos_prev=pos_prev)


def _sc_scatter(data, pos_cur, out_rows):
    return _sc_reorder(data, pos_cur, out_rows)


def _sc_scatter2(data, pos_a, pos_b, out_rows):
    T = pos_a.shape[1]
    D = data.shape[1]
    WIN = 64
    NSUB = 32
    PER = T // NSUB
    NW = PER // WIN

    def body(data_hbm, pa_hbm, pb_hbm, oa_hbm, ob_hbm,
             buf, pidx, gsem, asem, bsem):
        c = jax.lax.axis_index("c")
        s = jax.lax.axis_index("s")
        base = (c * 16 + s) * PER
        pltpu.sync_copy(pa_hbm.at[0, pl.ds(base, PER)], pidx.at[0])
        pltpu.sync_copy(pb_hbm.at[0, pl.ds(base, PER)], pidx.at[1])

        def gather_copy(w):
            src = data_hbm.at[pl.ds(base + w * WIN, WIN), :]
            return pltpu.make_async_copy(src, buf.at[w % 2], gsem.at[w % 2])

        def scatter_copy(w, o_hbm, row, sem):
            dst = o_hbm.at[pidx.at[row, pl.ds(w * WIN, WIN)]]
            return pltpu.make_async_copy(buf.at[w % 2], dst, sem.at[w % 2])

        g = [gather_copy(w) for w in range(NW)]
        sa = [scatter_copy(w, oa_hbm, 0, asem) for w in range(NW)]
        sb = [scatter_copy(w, ob_hbm, 1, bsem) for w in range(NW)]
        g[0].start()
        if NW > 1:
            g[1].start()
        for w in range(NW):
            g[w].wait()
            sa[w].start()
            sb[w].start()
            if w + 2 < NW:
                sa[w].wait()
                sb[w].wait()
                g[w + 2].start()
        for w in range(max(0, NW - 2), NW):
            sa[w].wait()
            sb[w].wait()

    out = jax.ShapeDtypeStruct((out_rows, D), data.dtype)
    scratch = [pltpu.VMEM((2, WIN, D), data.dtype),
               pltpu.VMEM((2, PER), jnp.int32),
               pltpu.SemaphoreType.DMA((2,)),
               pltpu.SemaphoreType.DMA((2,)),
               pltpu.SemaphoreType.DMA((2,))]
    k = pl.kernel(body, out_type=[out, out],
                  mesh=_vector_mesh(), scratch_types=scratch)
    return k(data, pos_a, pos_b)


def _sc_gather(data, pos):
    T = pos.shape[1]
    D = data.shape[1]
    WIN = 128

    @pl.kernel(out_type=jax.ShapeDtypeStruct((T, D), data.dtype),
               mesh=_vector_mesh())
    def k(data_hbm, p_hbm, o_hbm):
        def body(p_vmem, o_vmem):
            pltpu.sync_copy(data_hbm.at[p_vmem.at[0]], o_vmem)

        pltpu.emit_pipeline(
            body,
            grid=(T // WIN,),
            in_specs=[pl.BlockSpec((1, WIN), lambda i: (0, i))],
            out_specs=[pl.BlockSpec((WIN, D), lambda i: (i, 0))],
            core_axis_name=("c", "s"),
            dimension_semantics=(pltpu.PARALLEL,),
        )(p_hbm, o_hbm)

    return k(data, pos)


def _tc_pair(A_list, WA_list, bA, WB, bB, meta, reluB):
    NA = WA_list[0].shape[2]
    NB = WB.shape[2]
    nA = len(A_list)
    nb = TP // TM
    nsub = TM // SEG

    def body(*refs):
        eA_r, eB_r, act_r = refs[:3]
        a_refs = refs[3:3 + nA]
        wa_refs = refs[3 + nA:3 + 2 * nA]
        bA_r, wB_r, bB_r, o_ref = refs[3 + 2 * nA:]
        m = pl.program_id(0)

        for sub in range(nsub):
            s = m * nsub + sub
            rows = pl.ds(sub * SEG, SEG)

            @pl.when(act_r[s] == 1)
            def _():
                eA = eA_r[s]
                eB = eB_r[s]
                a0 = a_refs[0][rows, :].astype(jnp.bfloat16)
                acc = jnp.dot(a0, wa_refs[0][eA],
                              preferred_element_type=jnp.float32)
                for a_r, w_r in zip(a_refs[1:], wa_refs[1:]):
                    aj = a_r[rows, :].astype(jnp.bfloat16)
                    acc = acc + jnp.dot(aj, w_r[eA],
                                        preferred_element_type=jnp.float32)
                hA = jnp.maximum(acc + bA_r[...], 0.0).astype(jnp.bfloat16)
                out = jnp.dot(hA, wB_r[eB],
                              preferred_element_type=jnp.float32)
                out = out + bB_r[...]
                if reluB:
                    out = jnp.maximum(out, 0.0)
                o_ref[rows, :] = out.astype(o_ref.dtype)

    in_specs = []
    for A in A_list:
        K = A.shape[1]
        in_specs.append(pl.BlockSpec((TM, K), lambda m, *s: (m, 0)))
    for W in WA_list:
        in_specs.append(pl.BlockSpec(W.shape, lambda m, *s: (0, 0, 0)))
    in_specs.append(pl.BlockSpec((1, NA), lambda m, *s: (0, 0)))
    in_specs.append(pl.BlockSpec(WB.shape, lambda m, *s: (0, 0, 0)))
    in_specs.append(pl.BlockSpec((1, NB), lambda m, *s: (0, 0)))

    grid_spec = pltpu.PrefetchScalarGridSpec(
        num_scalar_prefetch=3,
        grid=(nb,),
        in_specs=in_specs,
        out_specs=pl.BlockSpec((TM, NB), lambda m, *s: (m, 0)),
    )
    return pl.pallas_call(
        body,
        grid_spec=grid_spec,
        out_shape=jax.ShapeDtypeStruct((TP, NB), jnp.float32),
    )(meta["eidA"], meta["eidB"], meta["act"],
      *A_list, *WA_list, bA, WB, bB)


def _routing_pairs(xyz_f):
    T = xyz_f.shape[0]
    NBK = NEXP * NEXP
    BLK = 128
    NBLK = T // BLK
    nsub = TP // SEG
    tril = jnp.tril(jnp.ones((BLK, BLK), jnp.float32), k=-1)
    bids = jnp.arange(NBK, dtype=jnp.int32)
    s_starts = jnp.arange(nsub, dtype=jnp.int32) * SEG
    metas = []
    for p in range(NL // 2):
        tids = []
        for i in (2 * p, 2 * p + 1):
            alpha = 2.0 ** (i + 1)
            t3 = jnp.floor(alpha * xyz_f).astype(jnp.int32) % NPD
            tids.append(t3[:, 0] + NPD * t3[:, 1] + NPD ** 2 * t3[:, 2])
        key = tids[0] * NEXP + tids[1]
        oh = (key[:, None] == bids[None, :]).astype(jnp.float32)
        oh3 = oh.reshape(NBLK, BLK, NBK)
        intra = jnp.einsum("lk,bkc->blc", tril, oh3)
        bs = jnp.sum(oh3, axis=1)
        blockoff = jnp.cumsum(bs, axis=0) - bs
        counts = jnp.sum(bs, axis=0).astype(jnp.int32)
        pc = (counts + SEG - 1) // SEG * SEG
        pcum = jnp.cumsum(pc)
        poffs = (pcum - pc).astype(jnp.float32)
        pos3 = intra + blockoff[:, None, :] + poffs[None, None, :]
        pos = jnp.sum(pos3 * oh3, axis=2).reshape(T).astype(jnp.int32)
        bk = jnp.sum(
            (s_starts[:, None] >= pcum[None, :]).astype(jnp.int32), axis=1)
        act = (s_starts < pcum[-1]).astype(jnp.int32)
        bk = jnp.minimum(bk, NBK - 1)
        metas.append(dict(
            pos=pos.reshape(1, T),
            eidA=bk // NEXP,
            eidB=bk % NEXP,
            act=act))
    return metas


def _to32(a):
    T, D = a.shape
    return jax.lax.bitcast_convert_type(
        a.reshape(T, D // 2, 2), jnp.int32)


def _from32(a, D):
    return jax.lax.bitcast_convert_type(
        a, jnp.bfloat16).reshape(a.shape[0], D)


def _pad_cols(a, to):
    pad = to - a.shape[-1]
    if pad == 0:
        return a
    cfg = [(0, 0)] * (a.ndim - 1) + [(0, pad)]
    return jnp.pad(a, cfg)


def _pad_rows(w, to):
    pad = to - w.shape[1]
    if pad == 0:
        return w
    return jnp.pad(w, [(0, 0), (0, pad), (0, 0)])


def kernel(lat, xyz, W0, W1, W2, W3, W4, W5, W6, W7,
           b0, b1, b2, b3, b4, b5, b6, b7):
    B, N, _ = xyz.shape
    T = B * N
    batch_shape = xyz.shape[:-1]
    XF = LATENT + IN_DIM
    XP = 384
    SKIP = HID - XF
    SKIPP = 256

    xyz_f = xyz.reshape(T, IN_DIM)
    x = jnp.concatenate(
        [jnp.broadcast_to(lat, batch_shape + (LATENT,)), xyz],
        axis=-1).reshape(T, XF)
    x = _pad_cols(x, XP)

    m0, m1, m2, m3 = _routing_pairs(xyz_f)

    bf = jnp.bfloat16
    W0p = _pad_rows(W0, XP).astype(bf)
    W1b = W1.astype(bf)
    W2b = W2.astype(bf)
    W3p = _pad_cols(W3, SKIPP).astype(bf)
    b3p = _pad_cols(b3, SKIPP)
    W4a = _pad_rows(W4[:, :SKIP, :], SKIPP).astype(bf)
    W4b = _pad_rows(W4[:, SKIP:, :], XP).astype(bf)
    W5b = W5.astype(bf)
    W6b = W6.astype(bf)
    W7p = _pad_cols(W7, 128).astype(bf)
    b7p = _pad_cols(b7, 128)

    x_s0 = _sc_scatter(x, m0["pos"], TP)
    h1 = _tc_pair([x_s0], [W0p], b0, W1b, b1, m0, reluB=True)
    h1p = _sc_permute(h1, m0["pos"], m1["pos"], TP)
    h3 = _tc_pair([h1p], [W2b], b2, W3p, b3p, m1, reluB=True)
    h3p = _sc_permute(h3, m1["pos"], m2["pos"], TP)
    x_s4 = _sc_scatter(x, m2["pos"], TP)
    h5 = _tc_pair([h3p, x_s4], [W4a, W4b], b4, W5b, b5, m2, reluB=True)
    h5p = _sc_permute(h5, m2["pos"], m3["pos"], TP)
    out7 = _tc_pair([h5p], [W6b], b6, W7p, b7p, m3, reluB=False)
    y = _sc_gather(out7, m3["pos"])
    return y[:, :OUT_DIM].astype(jnp.float32).reshape(
        batch_shape + (OUT_DIM,))
